# baseline (speedup 1.0000x reference)
import jax
import jax.numpy as jnp
from jax.experimental import pallas as pl
from jax.experimental.pallas import tpu as pltpu

_B, _C, _H, _W = 16, 3, 512, 512
_N = float(_B * _H * _W)
_GRID = 2
_BB = _B // _GRID
_ROWS = _BB * _H


def _bce_body(p_ref, g_ref, out_ref, acc_ref):
    i = pl.program_id(0)

    @pl.when(i == 0)
    def _init():
        acc_ref[...] = jnp.zeros_like(acc_ref)

    p = p_ref[:, 0].reshape(_ROWS, _W)
    g = g_ref[:, 0].reshape(_ROWS, _W)
    log2e = 1.4426950408889634
    q2 = p * (log2e - (2.0 * log2e) * g)
    z = jnp.exp2(-jnp.abs(q2))
    u2 = jnp.log2(1.0 + z)
    t = jnp.minimum(jnp.maximum(q2, 0.0) + u2, 100.0 * log2e)

    tb = t.astype(jnp.bfloat16)
    gb = g.astype(jnp.bfloat16)
    gtb = tb * gb
    ones = jnp.ones((1, _ROWS), jnp.bfloat16)
    dims = (((1,), (0,)), ((), ()))
    st = jax.lax.dot_general(ones, tb, dims, preferred_element_type=jnp.float32)
    sgt = jax.lax.dot_general(ones, gtb, dims, preferred_element_type=jnp.float32)
    sg = jax.lax.dot_general(ones, gb, dims, preferred_element_type=jnp.float32)
    acc_ref[0:1] += st
    acc_ref[1:2] += sgt
    acc_ref[2:3] += sg

    @pl.when(i == _GRID - 1)
    def _finish():
        ln2 = 0.6931471805599453
        sum_all = jnp.sum(acc_ref[0]) * ln2
        sum1 = jnp.sum(acc_ref[1]) * ln2
        n1 = jnp.sum(acc_ref[2])
        sum0 = sum_all - sum1
        n0 = _N - n1
        loss1 = sum1 / jnp.maximum(n1, 1.0)
        loss0 = sum0 / jnp.maximum(n0, 1.0)
        out_ref[0, 0] = loss1 + 0.5 * loss0


@jax.jit
def kernel(predict, ground):
    spec = pl.BlockSpec((_BB, 1, _H, _W), lambda i: (i, 0, 0, 0))
    out = pl.pallas_call(
        _bce_body,
        grid=(_GRID,),
        in_specs=[spec, spec],
        out_specs=pl.BlockSpec(memory_space=pltpu.SMEM),
        out_shape=jax.ShapeDtypeStruct((1, 1), jnp.float32),
        scratch_shapes=[pltpu.VMEM((8, _W), jnp.float32)],
        compiler_params=pltpu.CompilerParams(
            dimension_semantics=("arbitrary",),
        ),
    )(predict, ground)
    return out[0, 0]


# speedup vs baseline: 1.0787x; 1.0787x over previous
import jax
import jax.numpy as jnp
from jax.experimental import pallas as pl
from jax.experimental.pallas import tpu as pltpu

_B, _C, _H, _W = 16, 3, 512, 512
_N = float(_B * _H * _W)
_GRID = 4
_BB = _B // _GRID
_ROWS = _BB * _H


def _bce_body(p_ref, g_ref, out_ref, acc_ref):
    i = pl.program_id(0)

    @pl.when(i == 0)
    def _init():
        acc_ref[...] = jnp.zeros_like(acc_ref)

    p = p_ref[:, 0].reshape(_ROWS, _W)
    g = g_ref[:, 0].reshape(_ROWS, _W)
    log2e = 1.4426950408889634
    q2 = p * (log2e - (2.0 * log2e) * g)
    z = jnp.exp2(-jnp.abs(q2))
    u2 = jnp.log2(1.0 + z)
    t = jnp.minimum(jnp.maximum(q2, 0.0) + u2, 100.0 * log2e)

    tb = t.astype(jnp.bfloat16)
    gb = g.astype(jnp.bfloat16)
    gtb = tb * gb
    ones = jnp.ones((1, _ROWS), jnp.bfloat16)
    dims = (((1,), (0,)), ((), ()))
    st = jax.lax.dot_general(ones, tb, dims, preferred_element_type=jnp.float32)
    sgt = jax.lax.dot_general(ones, gtb, dims, preferred_element_type=jnp.float32)
    sg = jax.lax.dot_general(ones, gb, dims, preferred_element_type=jnp.float32)
    acc_ref[0:1] += st
    acc_ref[1:2] += sgt
    acc_ref[2:3] += sg

    @pl.when(i == _GRID - 1)
    def _finish():
        ln2 = 0.6931471805599453
        sum_all = jnp.sum(acc_ref[0]) * ln2
        sum1 = jnp.sum(acc_ref[1]) * ln2
        n1 = jnp.sum(acc_ref[2])
        sum0 = sum_all - sum1
        n0 = _N - n1
        loss1 = sum1 / jnp.maximum(n1, 1.0)
        loss0 = sum0 / jnp.maximum(n0, 1.0)
        out_ref[0, 0] = loss1 + 0.5 * loss0


@jax.jit
def kernel(predict, ground):
    spec = pl.BlockSpec((_BB, 1, _H, _W), lambda i: (i, 0, 0, 0))
    out = pl.pallas_call(
        _bce_body,
        grid=(_GRID,),
        in_specs=[spec, spec],
        out_specs=pl.BlockSpec(memory_space=pltpu.SMEM),
        out_shape=jax.ShapeDtypeStruct((1, 1), jnp.float32),
        scratch_shapes=[pltpu.VMEM((8, _W), jnp.float32)],
        compiler_params=pltpu.CompilerParams(
            dimension_semantics=("arbitrary",),
        ),
    )(predict, ground)
    return out[0, 0]


# speedup vs baseline: 1.1319x; 1.0494x over previous
import jax
import jax.numpy as jnp
from jax.experimental import pallas as pl
from jax.experimental.pallas import tpu as pltpu

_B, _C, _H, _W = 16, 3, 512, 512
_N = float(_B * _H * _W)
_GRID = 4
_BB = _B // _GRID
_ROWS = _BB * _H


def _bce_body(p_ref, g_ref, out_ref, acc_ref):
    i = pl.program_id(0)

    @pl.when(i == 0)
    def _init():
        acc_ref[...] = jnp.zeros_like(acc_ref)

    p = p_ref[:, 0].reshape(_ROWS, _W)
    g = g_ref[:, 0].reshape(_ROWS, _W)
    log2e = 1.4426950408889634
    q2 = p * (log2e - (2.0 * log2e) * g)
    neg_abs = jax.lax.bitcast_convert_type(
        jax.lax.bitcast_convert_type(q2, jnp.int32) | jnp.int32(-(2**31)),
        jnp.float32,
    )
    z = jnp.exp2(neg_abs)
    u2 = jnp.log2(1.0 + z)
    qb = q2.astype(jnp.bfloat16)
    ub = u2.astype(jnp.bfloat16)
    tb = jnp.minimum(
        jnp.maximum(qb, jnp.bfloat16(0.0)) + ub,
        jnp.bfloat16(100.0 * log2e),
    )
    gb = g.astype(jnp.bfloat16)
    gtb = tb * gb
    ones = jnp.ones((1, _ROWS), jnp.bfloat16)
    dims = (((1,), (0,)), ((), ()))
    st = jax.lax.dot_general(ones, tb, dims, preferred_element_type=jnp.float32)
    sgt = jax.lax.dot_general(ones, gtb, dims, preferred_element_type=jnp.float32)
    sg = jax.lax.dot_general(ones, gb, dims, preferred_element_type=jnp.float32)
    acc_ref[0:1] += st
    acc_ref[1:2] += sgt
    acc_ref[2:3] += sg

    @pl.when(i == _GRID - 1)
    def _finish():
        ln2 = 0.6931471805599453
        sum_all = jnp.sum(acc_ref[0]) * ln2
        sum1 = jnp.sum(acc_ref[1]) * ln2
        n1 = jnp.sum(acc_ref[2])
        sum0 = sum_all - sum1
        n0 = _N - n1
        loss1 = sum1 / jnp.maximum(n1, 1.0)
        loss0 = sum0 / jnp.maximum(n0, 1.0)
        out_ref[0, 0] = loss1 + 0.5 * loss0


@jax.jit
def kernel(predict, ground):
    spec = pl.BlockSpec((_BB, 1, _H, _W), lambda i: (i, 0, 0, 0))
    out = pl.pallas_call(
        _bce_body,
        grid=(_GRID,),
        in_specs=[spec, spec],
        out_specs=pl.BlockSpec(memory_space=pltpu.SMEM),
        out_shape=jax.ShapeDtypeStruct((1, 1), jnp.float32),
        scratch_shapes=[pltpu.VMEM((8, _W), jnp.float32)],
        compiler_params=pltpu.CompilerParams(
            dimension_semantics=("arbitrary",),
        ),
    )(predict, ground)
    return out[0, 0]


# speedup vs baseline: 1.1660x; 1.0301x over previous
import jax
import jax.numpy as jnp
from jax.experimental import pallas as pl
from jax.experimental.pallas import tpu as pltpu

_B, _C, _H, _W = 16, 3, 512, 512
_N = float(_B * _H * _W)
_GRID = 4
_BB = _B // _GRID
_ROWS = _BB * _H


def _bce_body(p_ref, g_ref, out_ref, acc_ref):
    i = pl.program_id(0)

    @pl.when(i == 0)
    def _init():
        acc_ref[...] = jnp.zeros_like(acc_ref)

    p = p_ref[:, 0].reshape(_ROWS, _W)
    g = g_ref[:, 0].reshape(_ROWS, _W)
    pi = jax.lax.bitcast_convert_type(p, jnp.int32)
    gi = jax.lax.bitcast_convert_type(g, jnp.int32)
    q = jax.lax.bitcast_convert_type(pi ^ (gi << 8), jnp.float32)
    neg_abs = jax.lax.bitcast_convert_type(
        pi | jnp.int32(-(2**31)), jnp.float32
    )
    z = jnp.exp(neg_abs)
    u = jnp.log(1.0 + z)
    qb = q.astype(jnp.bfloat16)
    ub = u.astype(jnp.bfloat16)
    tb = jnp.minimum(
        jnp.maximum(qb, jnp.bfloat16(0.0)) + ub,
        jnp.bfloat16(100.0),
    )
    gb = g.astype(jnp.bfloat16)
    gtb = tb * gb
    ones = jnp.ones((1, _ROWS), jnp.bfloat16)
    dims = (((1,), (0,)), ((), ()))
    st = jax.lax.dot_general(ones, tb, dims, preferred_element_type=jnp.float32)
    sgt = jax.lax.dot_general(ones, gtb, dims, preferred_element_type=jnp.float32)
    sg = jax.lax.dot_general(ones, gb, dims, preferred_element_type=jnp.float32)
    acc_ref[0:1] += st
    acc_ref[1:2] += sgt
    acc_ref[2:3] += sg

    @pl.when(i == _GRID - 1)
    def _finish():
        sum_all = jnp.sum(acc_ref[0])
        sum1 = jnp.sum(acc_ref[1])
        n1 = jnp.sum(acc_ref[2])
        sum0 = sum_all - sum1
        n0 = _N - n1
        loss1 = sum1 / jnp.maximum(n1, 1.0)
        loss0 = sum0 / jnp.maximum(n0, 1.0)
        out_ref[0, 0] = loss1 + 0.5 * loss0


@jax.jit
def kernel(predict, ground):
    spec = pl.BlockSpec((_BB, 1, _H, _W), lambda i: (i, 0, 0, 0))
    out = pl.pallas_call(
        _bce_body,
        grid=(_GRID,),
        in_specs=[spec, spec],
        out_specs=pl.BlockSpec(memory_space=pltpu.SMEM),
        out_shape=jax.ShapeDtypeStruct((1, 1), jnp.float32),
        scratch_shapes=[pltpu.VMEM((8, _W), jnp.float32)],
        compiler_params=pltpu.CompilerParams(
            dimension_semantics=("arbitrary",),
        ),
    )(predict, ground)
    return out[0, 0]
